# baseline (speedup 1.0000x reference)
import functools

import jax
import jax.numpy as jnp
from jax import lax
from jax.experimental import pallas as pl
from jax.experimental.pallas import tpu as pltpu
from jax.experimental.pallas import tpu_sc as plsc

N = 10000
D = 128
DH = 64
K = 50
ALPHA = 0.1
E = 320000
E2 = 327680
NP_ = 10240
ROWS_PW = NP_ // 16
NCHUNK = ROWS_PW // 128

EPW2 = E2 // 16
NJ2 = EPW2 // 128
EPW1 = E2 // 32
NJ1 = EPW1 // 128

_MESH = plsc.VectorSubcoreMesh(core_axis_name="c", subcore_axis_name="s")
_SC_PARAMS = pltpu.CompilerParams(use_tc_tiling_on_sc=False)


def _zero_vmem_2d(ref, rows, lanes):
    z = jnp.zeros((16,), jnp.float32)

    def body(r, _):
        for cc in range(lanes // 16):
            ref[r, pl.ds(cc * 16, 16)] = z
        return 0

    lax.fori_loop(0, rows, body, 0)


@functools.partial(
    pl.kernel,
    out_type=jax.ShapeDtypeStruct((2 * NP_, 16), jnp.float32),
    mesh=_MESH,
    compiler_params=_SC_PARAMS,
    scratch_types=[
        pltpu.VMEM((NJ1, 128), jnp.int32),
        pltpu.VMEM((128, 16), jnp.float32),
        pltpu.VMEM((128, 16), jnp.float32),
        pltpu.VMEM((128, 16), jnp.float32),
        pltpu.VMEM_SHARED((NP_, 16), jnp.float32),
    ],
)
def _deg_kernel(idst_hbm, ones_hbm, degp_hbm, vdst, onesv, zbuf, obuf, accd):
    c = lax.axis_index("c")
    s = lax.axis_index("s")
    w = c * 16 + s
    row0 = s * ROWS_PW
    coff = c * NP_

    pltpu.sync_copy(idst_hbm.at[w], vdst)
    pltpu.sync_copy(ones_hbm, onesv)
    _zero_vmem_2d(zbuf, 128, 16)
    for t in range(NCHUNK):
        pltpu.sync_copy(zbuf, accd.at[pl.ds(row0 + t * 128, 128)])
    plsc.subcore_barrier()

    def ch(j, _):
        pltpu.sync_copy(onesv, accd.at[vdst.at[j]], add=True)
        return 0

    lax.fori_loop(0, NJ1, ch, 0)
    plsc.subcore_barrier()

    for t in range(NCHUNK):
        pltpu.sync_copy(accd.at[pl.ds(row0 + t * 128, 128)], obuf)
        pltpu.sync_copy(obuf, degp_hbm.at[pl.ds(coff + row0 + t * 128, 128)])


@functools.partial(
    pl.kernel,
    out_type=(
        jax.ShapeDtypeStruct((2 * NP_, DH), jnp.float32),
        jax.ShapeDtypeStruct((2 * NP_, DH), jnp.float32),
    ),
    mesh=_MESH,
    compiler_params=_SC_PARAMS,
    scratch_types=[
        [pltpu.VMEM((8, 128), jnp.int32)] * 2,
        [pltpu.VMEM((8, 128), jnp.int32)] * 2,
        [pltpu.VMEM((128, DH), jnp.float32)] * 8,
        pltpu.VMEM_SHARED((NP_, DH), jnp.float32),
        pltpu.SemaphoreType.DMA((8,)),
        pltpu.SemaphoreType.DMA((8,)),
        pltpu.SemaphoreType.DMA((4,)),
    ],
)
def _prop_kernel(y0_hbm, hh_hbm, s0_hbm, isrc_hbm, idst_hbm,
                 ya_hbm, yb_hbm,
                 visrc, vidst, gb, acc, gsem, ssem, isem):
    c = lax.axis_index("c")
    s = lax.axis_index("s")
    row0 = s * ROWS_PW
    coff = c * NP_
    abuf, ybuf, hbuf, sbuf, obuf, zbuf = gb[0], gb[1], gb[2], gb[3], gb[4], gb[5]

    for t in range(NCHUNK):
        r0 = coff + row0 + t * 128
        pltpu.sync_copy(y0_hbm.at[pl.ds(r0, 128)], abuf)
        pltpu.sync_copy(abuf, ya_hbm.at[pl.ds(r0, 128)])
    plsc.subcore_barrier()

    def one_round(ysrc, ydst):
        _zero_vmem_2d(zbuf, 128, DH)
        for t in range(NCHUNK):
            pltpu.sync_copy(zbuf, acc.at[pl.ds(row0 + t * 128, 128)])
        plsc.subcore_barrier()

        def do_block(vs, vd):
            gd = [
                pltpu.async_copy(ysrc.at[vs.at[b]], gb[b], gsem.at[b])
                for b in range(8)
            ]
            sd = []
            for b in range(8):
                gd[b].wait()
                sd.append(pltpu.async_copy(
                    gb[b], acc.at[vd.at[b]], ssem.at[b], add=True))
            for d in sd:
                d.wait()

        pltpu.sync_copy(isrc_hbm.at[c, s, pl.ds(0, 8)], visrc[0])
        pltpu.sync_copy(idst_hbm.at[s, pl.ds(0, 8)], vidst[0])

        def block_pair(pp, _):
            j0 = pp * 16
            ib = [
                pltpu.async_copy(isrc_hbm.at[c, s, pl.ds(j0 + 8, 8)],
                                 visrc[1], isem.at[0]),
                pltpu.async_copy(idst_hbm.at[s, pl.ds(j0 + 8, 8)],
                                 vidst[1], isem.at[1]),
            ]
            do_block(visrc[0], vidst[0])
            for d in ib:
                d.wait()
            ia = [
                pltpu.async_copy(isrc_hbm.at[c, s, pl.ds(j0 + 16, 8)],
                                 visrc[0], isem.at[2]),
                pltpu.async_copy(idst_hbm.at[s, pl.ds(j0 + 16, 8)],
                                 vidst[0], isem.at[3]),
            ]
            do_block(visrc[1], vidst[1])
            for d in ia:
                d.wait()
            return 0

        lax.fori_loop(0, NJ2 // 16, block_pair, 0)
        plsc.subcore_barrier()

        def comb(t, _):
            r0 = row0 + t * 128
            pltpu.sync_copy(acc.at[pl.ds(r0, 128)], abuf)
            pltpu.sync_copy(ysrc.at[pl.ds(coff + r0, 128)], ybuf)
            pltpu.sync_copy(hh_hbm.at[pl.ds(coff + r0, 128)], hbuf)
            pltpu.sync_copy(s0_hbm.at[pl.ds(coff + r0, 128)], sbuf)

            def row(r, _):
                for cc in range(DH // 16):
                    sl = pl.ds(cc * 16, 16)
                    obuf[r, sl] = (abuf[r, sl] + ybuf[r, sl]) * sbuf[r, sl] \
                        + hbuf[r, sl]
                return 0

            lax.fori_loop(0, 128, row, 0)
            pltpu.sync_copy(obuf, ydst.at[pl.ds(coff + r0, 128)])
            return 0

        lax.fori_loop(0, NCHUNK, comb, 0)
        plsc.subcore_barrier()

    def two_rounds(i, _):
        one_round(ya_hbm, yb_hbm)
        one_round(yb_hbm, ya_hbm)
        return 0

    lax.fori_loop(0, K // 2, two_rounds, 0)


def _lin_body(x_ref, w_ref, b_ref, o_ref):
    o_ref[...] = jnp.maximum(
        jnp.dot(x_ref[...], w_ref[...], preferred_element_type=jnp.float32)
        + b_ref[...],
        0.0,
    )


def _linear_relu(x, W, b):
    return pl.pallas_call(
        _lin_body,
        grid=(10,),
        in_specs=[
            pl.BlockSpec((1000, D), lambda i: (i, 0)),
            pl.BlockSpec((D, D), lambda i: (0, 0)),
            pl.BlockSpec((1, D), lambda i: (0, 0)),
        ],
        out_specs=pl.BlockSpec((1000, D), lambda i: (i, 0)),
        out_shape=jax.ShapeDtypeStruct((N, D), jnp.float32),
    )(x, W, b.reshape(1, D))


@jax.jit
def kernel(x, edge_index, W, b):
    h = _linear_relu(x, W, b)

    src = edge_index[0].astype(jnp.int32)
    dst = edge_index[1].astype(jnp.int32)
    perm = jnp.argsort(src)
    src = src[perm]
    dst = dst[perm]
    pad = E2 - E
    srcp = jnp.concatenate([src, jnp.full((pad,), N, jnp.int32)])
    dstp = jnp.concatenate([dst, jnp.full((pad,), N, jnp.int32)])

    idst1 = dstp.reshape(32, NJ1, 128)
    zpadc = jnp.zeros((16, 8, 128), jnp.int32)
    isrc2 = jnp.concatenate([srcp.reshape(16, NJ2, 128), zpadc], axis=1)
    isrc2 = jnp.stack([isrc2, isrc2 + NP_])
    idst2 = jnp.concatenate([dstp.reshape(16, NJ2, 128), zpadc], axis=1)

    ones = jnp.ones((128, 16), jnp.float32)
    degp = _deg_kernel(idst1, ones)
    deg = degp[:NP_, 0][:N] + degp[NP_:, 0][:N] + 1.0

    dinv = lax.rsqrt(deg)
    y0 = h * dinv[:, None]
    y0p = jnp.zeros((NP_, D), jnp.float32).at[:N].set(y0)
    y0f = jnp.concatenate([y0p[:, :DH], y0p[:, DH:]], axis=0)
    hhf = ALPHA * y0f
    s0 = jnp.zeros((NP_,), jnp.float32).at[:N].set((1.0 - ALPHA) / deg)
    s0f = jnp.tile(s0[:, None], (2, DH))

    ya, _ = _prop_kernel(y0f, hhf, s0f, isrc2, idst2)
    yfin = jnp.concatenate([ya[:N], ya[NP_:NP_ + N]], axis=1)
    return yfin * jnp.sqrt(deg)[:, None]


# speedup vs baseline: 1.9480x; 1.9480x over previous
import functools

import jax
import jax.numpy as jnp
from jax import lax
from jax.experimental import pallas as pl
from jax.experimental.pallas import tpu as pltpu
from jax.experimental.pallas import tpu_sc as plsc

N = 10000
D = 128
DH = 64
K = 50
ALPHA = 0.1
E = 320000
E2 = 327680
NP_ = 10240
ROWS_PW = NP_ // 16
NCHUNK = ROWS_PW // 128

EPW2 = E2 // 16
NJ2 = EPW2 // 128
EPW1 = E2 // 32
NJ1 = EPW1 // 128

_MESH = plsc.VectorSubcoreMesh(core_axis_name="c", subcore_axis_name="s")
_SC_PARAMS = pltpu.CompilerParams(use_tc_tiling_on_sc=False)


def _zero_vmem_2d(ref, rows, lanes):
    z = jnp.zeros((16,), jnp.float32)

    def body(r, _):
        for cc in range(lanes // 16):
            ref[r, pl.ds(cc * 16, 16)] = z
        return 0

    lax.fori_loop(0, rows, body, 0)


@functools.partial(
    pl.kernel,
    out_type=jax.ShapeDtypeStruct((2 * NP_, 16), jnp.float32),
    mesh=_MESH,
    compiler_params=_SC_PARAMS,
    scratch_types=[
        pltpu.VMEM((NJ1, 128), jnp.int32),
        pltpu.VMEM((128, 16), jnp.float32),
        pltpu.VMEM((128, 16), jnp.float32),
        pltpu.VMEM((128, 16), jnp.float32),
        pltpu.VMEM_SHARED((NP_, 16), jnp.float32),
    ],
)
def _deg_kernel(idst_hbm, ones_hbm, degp_hbm, vdst, onesv, zbuf, obuf, accd):
    c = lax.axis_index("c")
    s = lax.axis_index("s")
    w = c * 16 + s
    row0 = s * ROWS_PW
    coff = c * NP_

    pltpu.sync_copy(idst_hbm.at[w], vdst)
    pltpu.sync_copy(ones_hbm, onesv)
    _zero_vmem_2d(zbuf, 128, 16)
    for t in range(NCHUNK):
        pltpu.sync_copy(zbuf, accd.at[pl.ds(row0 + t * 128, 128)])
    plsc.subcore_barrier()

    def ch(j, _):
        pltpu.sync_copy(onesv, accd.at[vdst.at[j]], add=True)
        return 0

    lax.fori_loop(0, NJ1, ch, 0)
    plsc.subcore_barrier()

    for t in range(NCHUNK):
        pltpu.sync_copy(accd.at[pl.ds(row0 + t * 128, 128)], obuf)
        pltpu.sync_copy(obuf, degp_hbm.at[pl.ds(coff + row0 + t * 128, 128)])


@functools.partial(
    pl.kernel,
    out_type=jax.ShapeDtypeStruct((2 * NP_, DH), jnp.float32),
    mesh=_MESH,
    compiler_params=_SC_PARAMS,
    scratch_types=[
        [pltpu.VMEM((5, 128), jnp.int32)] * 2,
        [pltpu.VMEM((5, 128), jnp.int32)] * 2,
        [pltpu.VMEM((128, DH), jnp.float32)] * 5,
        pltpu.VMEM_SHARED((NP_, DH), jnp.float32),
        pltpu.VMEM_SHARED((NP_, DH), jnp.float32),
        pltpu.SemaphoreType.DMA((5,)),
        pltpu.SemaphoreType.DMA((5,)),
        pltpu.SemaphoreType.DMA((4,)),
    ],
)
def _prop_kernel(y0_hbm, hq_hbm, s0_hbm, isrc_hbm, idst_hbm,
                 yout_hbm,
                 visrc, vidst, gb, ybuf_s, pbuf_s, gsem, ssem, isem):
    c = lax.axis_index("c")
    s = lax.axis_index("s")
    row0 = s * ROWS_PW
    coff = c * NP_
    BLK = 5

    for t in range(NCHUNK):
        r0 = row0 + t * 128
        pltpu.sync_copy(y0_hbm.at[pl.ds(coff + r0, 128)], gb[0])
        pltpu.sync_copy(gb[0], ybuf_s.at[pl.ds(r0, 128)])
        pltpu.sync_copy(hq_hbm.at[pl.ds(coff + r0, 128)], gb[1])

        def irow(r, _):
            for cc in range(DH // 16):
                sl = pl.ds(cc * 16, 16)
                gb[2][r, sl] = gb[0][r, sl] + gb[1][r, sl]
            return 0

        lax.fori_loop(0, 128, irow, 0)
        pltpu.sync_copy(gb[2], pbuf_s.at[pl.ds(r0, 128)])
    plsc.subcore_barrier()

    def do_block(vs, vd):
        gd = [
            pltpu.async_copy(ybuf_s.at[vs.at[b]], gb[b], gsem.at[b])
            for b in range(BLK)
        ]
        sd = []
        for b in range(BLK):
            gd[b].wait()
            sd.append(pltpu.async_copy(
                gb[b], pbuf_s.at[vd.at[b]], ssem.at[b], add=True))
        for d in sd:
            d.wait()

    def one_round(k, _):
        pltpu.sync_copy(isrc_hbm.at[s, pl.ds(0, BLK)], visrc[0])
        pltpu.sync_copy(idst_hbm.at[s, pl.ds(0, BLK)], vidst[0])

        def block_pair(pp, _):
            j0 = pp * (2 * BLK)
            ib = [
                pltpu.async_copy(isrc_hbm.at[s, pl.ds(j0 + BLK, BLK)],
                                 visrc[1], isem.at[0]),
                pltpu.async_copy(idst_hbm.at[s, pl.ds(j0 + BLK, BLK)],
                                 vidst[1], isem.at[1]),
            ]
            do_block(visrc[0], vidst[0])
            for d in ib:
                d.wait()
            ia = [
                pltpu.async_copy(isrc_hbm.at[s, pl.ds(j0 + 2 * BLK, BLK)],
                                 visrc[0], isem.at[2]),
                pltpu.async_copy(idst_hbm.at[s, pl.ds(j0 + 2 * BLK, BLK)],
                                 vidst[0], isem.at[3]),
            ]
            do_block(visrc[1], vidst[1])
            for d in ia:
                d.wait()
            return 0

        lax.fori_loop(0, NJ2 // (2 * BLK), block_pair, 0)
        plsc.subcore_barrier()

        def comb(t, _):
            r0 = row0 + t * 128
            pltpu.sync_copy(pbuf_s.at[pl.ds(r0, 128)], gb[0])
            pltpu.sync_copy(s0_hbm.at[pl.ds(coff + r0, 128)], gb[1])
            pltpu.sync_copy(hq_hbm.at[pl.ds(coff + r0, 128)], gb[2])

            def row(r, _):
                for cc in range(DH // 16):
                    sl = pl.ds(cc * 16, 16)
                    yv = gb[0][r, sl] * gb[1][r, sl]
                    gb[3][r, sl] = yv
                    gb[4][r, sl] = yv + gb[2][r, sl]
                return 0

            lax.fori_loop(0, 128, row, 0)
            pltpu.sync_copy(gb[3], ybuf_s.at[pl.ds(r0, 128)])
            pltpu.sync_copy(gb[4], pbuf_s.at[pl.ds(r0, 128)])
            return 0

        lax.fori_loop(0, NCHUNK, comb, 0)
        plsc.subcore_barrier()
        return 0

    lax.fori_loop(0, K, one_round, 0)

    for t in range(NCHUNK):
        r0 = row0 + t * 128
        pltpu.sync_copy(ybuf_s.at[pl.ds(r0, 128)], gb[0])
        pltpu.sync_copy(gb[0], yout_hbm.at[pl.ds(coff + r0, 128)])


def _lin_body(x_ref, w_ref, b_ref, o_ref):
    o_ref[...] = jnp.maximum(
        jnp.dot(x_ref[...], w_ref[...], preferred_element_type=jnp.float32)
        + b_ref[...],
        0.0,
    )


def _linear_relu(x, W, b):
    return pl.pallas_call(
        _lin_body,
        grid=(10,),
        in_specs=[
            pl.BlockSpec((1000, D), lambda i: (i, 0)),
            pl.BlockSpec((D, D), lambda i: (0, 0)),
            pl.BlockSpec((1, D), lambda i: (0, 0)),
        ],
        out_specs=pl.BlockSpec((1000, D), lambda i: (i, 0)),
        out_shape=jax.ShapeDtypeStruct((N, D), jnp.float32),
    )(x, W, b.reshape(1, D))


@jax.jit
def kernel(x, edge_index, W, b):
    h = _linear_relu(x, W, b)

    src = edge_index[0].astype(jnp.int32)
    dst = edge_index[1].astype(jnp.int32)
    pad = E2 - E
    srcp = jnp.concatenate([src, jnp.full((pad,), N, jnp.int32)])
    dstp = jnp.concatenate([dst, jnp.full((pad,), N, jnp.int32)])

    idst1 = dstp.reshape(32, NJ1, 128)
    zpadc = jnp.zeros((16, 8, 128), jnp.int32)
    isrc2 = jnp.concatenate([srcp.reshape(16, NJ2, 128), zpadc], axis=1)
    idst2 = jnp.concatenate([dstp.reshape(16, NJ2, 128), zpadc], axis=1)

    ones = jnp.ones((128, 16), jnp.float32)
    degp = _deg_kernel(idst1, ones)
    deg = degp[:NP_, 0][:N] + degp[NP_:, 0][:N] + 1.0

    dinv = lax.rsqrt(deg)
    sq = jnp.sqrt(deg)
    y0 = h * dinv[:, None]
    y0p = jnp.zeros((NP_, D), jnp.float32).at[:N].set(y0)
    y0f = jnp.concatenate([y0p[:, :DH], y0p[:, DH:]], axis=0)
    hq = (ALPHA / (1.0 - ALPHA)) * sq[:, None] * h
    hqp = jnp.zeros((NP_, D), jnp.float32).at[:N].set(hq)
    hqf = jnp.concatenate([hqp[:, :DH], hqp[:, DH:]], axis=0)
    s0 = jnp.zeros((NP_,), jnp.float32).at[:N].set((1.0 - ALPHA) / deg)
    s0f = jnp.tile(s0[:, None], (2, DH))

    ya = _prop_kernel(y0f, hqf, s0f, isrc2, idst2)
    yfin = jnp.concatenate([ya[:N], ya[NP_:NP_ + N]], axis=1)
    return yfin * sq[:, None]


# speedup vs baseline: 2.0123x; 1.0330x over previous
import functools

import jax
import jax.numpy as jnp
from jax import lax
from jax.experimental import pallas as pl
from jax.experimental.pallas import tpu as pltpu
from jax.experimental.pallas import tpu_sc as plsc

N = 10000
D = 128
DH = 64
K = 50
ALPHA = 0.1
E = 320000
E2 = 327680
NP_ = 10240
ROWS_PW = NP_ // 16
NCHUNK = ROWS_PW // 128

EPW2 = E2 // 16
NJ2 = EPW2 // 128
EPW1 = E2 // 32
NJ1 = EPW1 // 128

_MESH = plsc.VectorSubcoreMesh(core_axis_name="c", subcore_axis_name="s")
_SC_PARAMS = pltpu.CompilerParams(use_tc_tiling_on_sc=False)


def _zero_vmem_2d(ref, rows, lanes):
    z = jnp.zeros((16,), jnp.float32)

    def body(r, _):
        for cc in range(lanes // 16):
            ref[r, pl.ds(cc * 16, 16)] = z
        return 0

    lax.fori_loop(0, rows, body, 0)


@functools.partial(
    pl.kernel,
    out_type=jax.ShapeDtypeStruct((2 * NP_, 16), jnp.float32),
    mesh=_MESH,
    compiler_params=_SC_PARAMS,
    scratch_types=[
        pltpu.VMEM((NJ1, 128), jnp.int32),
        pltpu.VMEM((128, 16), jnp.float32),
        pltpu.VMEM((128, 16), jnp.float32),
        pltpu.VMEM((128, 16), jnp.float32),
        pltpu.VMEM_SHARED((NP_, 16), jnp.float32),
    ],
)
def _deg_kernel(idst_hbm, ones_hbm, degp_hbm, vdst, onesv, zbuf, obuf, accd):
    c = lax.axis_index("c")
    s = lax.axis_index("s")
    w = c * 16 + s
    row0 = s * ROWS_PW
    coff = c * NP_

    pltpu.sync_copy(idst_hbm.at[w], vdst)
    pltpu.sync_copy(ones_hbm, onesv)
    _zero_vmem_2d(zbuf, 128, 16)
    for t in range(NCHUNK):
        pltpu.sync_copy(zbuf, accd.at[pl.ds(row0 + t * 128, 128)])
    plsc.subcore_barrier()

    def ch(j, _):
        pltpu.sync_copy(onesv, accd.at[vdst.at[j]], add=True)
        return 0

    lax.fori_loop(0, NJ1, ch, 0)
    plsc.subcore_barrier()

    for t in range(NCHUNK):
        pltpu.sync_copy(accd.at[pl.ds(row0 + t * 128, 128)], obuf)
        pltpu.sync_copy(obuf, degp_hbm.at[pl.ds(coff + row0 + t * 128, 128)])


@functools.partial(
    pl.kernel,
    out_type=jax.ShapeDtypeStruct((2 * NP_, DH), jnp.float32),
    mesh=_MESH,
    compiler_params=_SC_PARAMS,
    scratch_types=[
        [pltpu.VMEM((5, 128), jnp.int32)] * 2,
        [pltpu.VMEM((5, 128), jnp.int32)] * 2,
        [pltpu.VMEM((128, DH), jnp.float32)] * 5,
        pltpu.VMEM_SHARED((NP_, DH), jnp.float32),
        pltpu.VMEM_SHARED((NP_, DH), jnp.float32),
        pltpu.SemaphoreType.DMA((5,)),
        pltpu.SemaphoreType.DMA((5,)),
        pltpu.SemaphoreType.DMA((4,)),
    ],
)
def _prop_kernel(y0_hbm, hq_hbm, s0_hbm, isrc_hbm, idst_hbm,
                 yout_hbm,
                 visrc, vidst, gb, ybuf_s, pbuf_s, gsem, ssem, isem):
    c = lax.axis_index("c")
    s = lax.axis_index("s")
    row0 = s * ROWS_PW
    coff = c * NP_
    BLK = 5

    for t in range(NCHUNK):
        r0 = row0 + t * 128
        pltpu.sync_copy(y0_hbm.at[pl.ds(coff + r0, 128)], gb[0])
        pltpu.sync_copy(gb[0], ybuf_s.at[pl.ds(r0, 128)])
        pltpu.sync_copy(hq_hbm.at[pl.ds(coff + r0, 128)], gb[1])

        def irow(r, _):
            for cc in range(DH // 16):
                sl = pl.ds(cc * 16, 16)
                gb[2][r, sl] = gb[0][r, sl] + gb[1][r, sl]
            return 0

        lax.fori_loop(0, 128, irow, 0)
        pltpu.sync_copy(gb[2], pbuf_s.at[pl.ds(r0, 128)])
    plsc.subcore_barrier()

    def do_block(vs, vd):
        gd = [
            pltpu.async_copy(ybuf_s.at[vs.at[b]], gb[b], gsem.at[b])
            for b in range(BLK)
        ]
        sd = []
        for b in range(BLK):
            gd[b].wait()
            sd.append(pltpu.async_copy(
                gb[b], pbuf_s.at[vd.at[b]], ssem.at[b], add=True))
        for d in sd:
            d.wait()

    def one_round(k, _):
        pltpu.sync_copy(isrc_hbm.at[s, pl.ds(0, BLK)], visrc[0])
        pltpu.sync_copy(idst_hbm.at[s, pl.ds(0, BLK)], vidst[0])

        def block_pair(pp, _):
            j0 = pp * (2 * BLK)
            ib = [
                pltpu.async_copy(isrc_hbm.at[s, pl.ds(j0 + BLK, BLK)],
                                 visrc[1], isem.at[0]),
                pltpu.async_copy(idst_hbm.at[s, pl.ds(j0 + BLK, BLK)],
                                 vidst[1], isem.at[1]),
            ]
            do_block(visrc[0], vidst[0])
            for d in ib:
                d.wait()
            ia = [
                pltpu.async_copy(isrc_hbm.at[s, pl.ds(j0 + 2 * BLK, BLK)],
                                 visrc[0], isem.at[2]),
                pltpu.async_copy(idst_hbm.at[s, pl.ds(j0 + 2 * BLK, BLK)],
                                 vidst[0], isem.at[3]),
            ]
            do_block(visrc[1], vidst[1])
            for d in ia:
                d.wait()
            return 0

        lax.fori_loop(0, NJ2 // (2 * BLK), block_pair, 0)
        plsc.subcore_barrier()

        def comb(t, _):
            r0 = row0 + t * 128
            ld = [
                pltpu.async_copy(pbuf_s.at[pl.ds(r0, 128)], gb[0],
                                 gsem.at[0]),
                pltpu.async_copy(s0_hbm.at[pl.ds(coff + r0, 128)], gb[1],
                                 gsem.at[1]),
                pltpu.async_copy(hq_hbm.at[pl.ds(coff + r0, 128)], gb[2],
                                 gsem.at[2]),
            ]
            for d in ld:
                d.wait()

            def row(r, _):
                for cc in range(DH // 16):
                    sl = pl.ds(cc * 16, 16)
                    yv = gb[0][r, sl] * gb[1][r, sl]
                    gb[3][r, sl] = yv
                    gb[4][r, sl] = yv + gb[2][r, sl]
                return 0

            lax.fori_loop(0, 128, row, 0)
            st = [
                pltpu.async_copy(gb[3], ybuf_s.at[pl.ds(r0, 128)],
                                 ssem.at[0]),
                pltpu.async_copy(gb[4], pbuf_s.at[pl.ds(r0, 128)],
                                 ssem.at[1]),
            ]
            for d in st:
                d.wait()
            return 0

        lax.fori_loop(0, NCHUNK, comb, 0)
        plsc.subcore_barrier()
        return 0

    lax.fori_loop(0, K, one_round, 0)

    for t in range(NCHUNK):
        r0 = row0 + t * 128
        pltpu.sync_copy(ybuf_s.at[pl.ds(r0, 128)], gb[0])
        pltpu.sync_copy(gb[0], yout_hbm.at[pl.ds(coff + r0, 128)])


def _lin_body(x_ref, w_ref, b_ref, o_ref):
    o_ref[...] = jnp.maximum(
        jnp.dot(x_ref[...], w_ref[...], preferred_element_type=jnp.float32)
        + b_ref[...],
        0.0,
    )


def _linear_relu(x, W, b):
    return pl.pallas_call(
        _lin_body,
        grid=(10,),
        in_specs=[
            pl.BlockSpec((1000, D), lambda i: (i, 0)),
            pl.BlockSpec((D, D), lambda i: (0, 0)),
            pl.BlockSpec((1, D), lambda i: (0, 0)),
        ],
        out_specs=pl.BlockSpec((1000, D), lambda i: (i, 0)),
        out_shape=jax.ShapeDtypeStruct((N, D), jnp.float32),
    )(x, W, b.reshape(1, D))


@jax.jit
def kernel(x, edge_index, W, b):
    h = _linear_relu(x, W, b)

    src = edge_index[0].astype(jnp.int32)
    dst = edge_index[1].astype(jnp.int32)
    pad = E2 - E
    srcp = jnp.concatenate([src, jnp.full((pad,), N, jnp.int32)])
    dstp = jnp.concatenate([dst, jnp.full((pad,), N, jnp.int32)])

    idst1 = dstp.reshape(32, NJ1, 128)
    zpadc = jnp.zeros((16, 8, 128), jnp.int32)
    isrc2 = jnp.concatenate([srcp.reshape(16, NJ2, 128), zpadc], axis=1)
    idst2 = jnp.concatenate([dstp.reshape(16, NJ2, 128), zpadc], axis=1)

    ones = jnp.ones((128, 16), jnp.float32)
    degp = _deg_kernel(idst1, ones)
    deg = degp[:NP_, 0][:N] + degp[NP_:, 0][:N] + 1.0

    dinv = lax.rsqrt(deg)
    sq = jnp.sqrt(deg)
    y0 = h * dinv[:, None]
    y0p = jnp.zeros((NP_, D), jnp.float32).at[:N].set(y0)
    y0f = jnp.concatenate([y0p[:, :DH], y0p[:, DH:]], axis=0)
    hq = (ALPHA / (1.0 - ALPHA)) * sq[:, None] * h
    hqp = jnp.zeros((NP_, D), jnp.float32).at[:N].set(hq)
    hqf = jnp.concatenate([hqp[:, :DH], hqp[:, DH:]], axis=0)
    s0 = jnp.zeros((NP_,), jnp.float32).at[:N].set((1.0 - ALPHA) / deg)
    s0f = jnp.tile(s0[:, None], (2, DH))

    ya = _prop_kernel(y0f, hqf, s0f, isrc2, idst2)
    yfin = jnp.concatenate([ya[:N], ya[NP_:NP_ + N]], axis=1)
    return yfin * sq[:, None]
